# baseline (speedup 1.0000x reference)
import functools

import jax
import jax.numpy as jnp
from jax import lax
from jax.experimental import pallas as pl
from jax.experimental.pallas import tpu as pltpu
from jax.experimental.pallas import tpu_sc as plsc

N = 10000
E = 320000
D = 128
H = 256
C = 47

NC = 2
NS = 16
CW = 125
WCH = 16
NWIN = 10
CHUNKS = WCH * NWIN
EPAD = NS * CHUNKS * CW
NPAD = 10112
RPT = NPAD // NS

BN = 1000


NBLK = N // BN


def _pay_from(v, gmax, tv):
    m = jnp.maximum(v, 0.0) + 1e-7
    ex = jnp.exp(m * tv - gmax)
    mex = m * ex
    hd = D // 2
    return (jnp.concatenate([mex[:, :hd], ex[:, :hd]], axis=1),
            jnp.concatenate([mex[:, hd:], ex[:, hd:]], axis=1))


def _enc_body(x_ref, w_ref, b_ref, t_ref, h_ref, pay_ref, hs_scr, gmax_scr):
    j = pl.program_id(0)

    @pl.when(j < NBLK)
    def _():
        h = jnp.dot(x_ref[...], w_ref[...], preferred_element_type=jnp.float32)
        h = h + b_ref[...]
        h_ref[...] = h
        hs_scr[pl.ds(j * BN, BN), :] = h
        m = jnp.maximum(h, 0.0) + 1e-7
        bm = jnp.max(m * t_ref[0, 0], axis=0, keepdims=True)

        @pl.when(j == 0)
        def _():
            gmax_scr[...] = bm

        @pl.when(j > 0)
        def _():
            gmax_scr[...] = jnp.maximum(gmax_scr[...], bm)

    @pl.when(j >= NBLK)
    def _():
        i = j - NBLK
        v = hs_scr[pl.ds(i * BN, BN), :]
        p0, p1 = _pay_from(v, gmax_scr[...], t_ref[0, 0])
        pay_ref[0] = p0
        pay_ref[1] = p1


def _enc(x, w, b, t0):
    return pl.pallas_call(
        _enc_body,
        grid=(2 * NBLK,),
        in_specs=[
            pl.BlockSpec((BN, D), lambda i: (jnp.minimum(i, NBLK - 1), 0)),
            pl.BlockSpec((D, D), lambda i: (0, 0)),
            pl.BlockSpec((1, D), lambda i: (0, 0)),
            pl.BlockSpec(memory_space=pltpu.SMEM),
        ],
        out_specs=[
            pl.BlockSpec((BN, D), lambda i: (jnp.minimum(i, NBLK - 1), 0)),
            pl.BlockSpec((2, BN, D),
                         lambda i: (0, jnp.maximum(i - NBLK, 0), 0)),
        ],
        out_shape=[
            jax.ShapeDtypeStruct((N, D), jnp.float32),
            jax.ShapeDtypeStruct((2, N, D), jnp.float32),
        ],
        scratch_shapes=[
            pltpu.VMEM((N, D), jnp.float32),
            pltpu.VMEM((1, D), jnp.float32),
        ],
    )(x, w, b.reshape(1, D), t0.reshape(1, 1))


def _conv_tail(acc0, acc1, v, hp, w1_ref, b1_ref, mg_ref, mb_ref, w2_ref,
               b2_ref, s_ref):
    hd = D // 2
    num = jnp.concatenate([acc0[:, :hd], acc1[:, :hd]], axis=1)
    den = jnp.concatenate([acc0[:, hd:], acc1[:, hd:]], axis=1)
    agg = num / (den + 1e-16)
    msg = agg / jnp.sqrt(jnp.sum(agg * agg, axis=-1, keepdims=True) + 1e-24)
    xn = jnp.sqrt(jnp.sum(v * v, axis=-1, keepdims=True) + 1e-24)
    out = v + msg * xn * s_ref[0, 0]
    z = jnp.dot(out, w1_ref[...], preferred_element_type=jnp.float32)
    z = z + b1_ref[...]
    mu = jnp.mean(z, axis=-1, keepdims=True)
    var = jnp.mean(jnp.square(z - mu), axis=-1, keepdims=True)
    z = (z - mu) / jnp.sqrt(var + 1e-5) * mg_ref[...] + mb_ref[...]
    z = jnp.maximum(z, 0.0)
    r = jnp.dot(z, w2_ref[...], preferred_element_type=jnp.float32)
    return hp + r + b2_ref[...]


def _post_mid_body(acc_ref0, acc_ref1, v_ref, hp_ref, w1_ref, b1_ref, mg_ref,
                   mb_ref, w2_ref, b2_ref, g_ref, be_ref, st_ref,
                   h_ref, vn_ref, pay_ref, vs_scr, gmax_scr):
    j = pl.program_id(0)

    @pl.when(j < NBLK)
    def _():
        h = _conv_tail(acc_ref0[0], acc_ref1[0], v_ref[...], hp_ref[...],
                       w1_ref, b1_ref, mg_ref, mb_ref, w2_ref, b2_ref, st_ref)
        h_ref[...] = h
        mu = jnp.mean(h, axis=-1, keepdims=True)
        var = jnp.mean(jnp.square(h - mu), axis=-1, keepdims=True)
        vn = jnp.maximum(
            (h - mu) / jnp.sqrt(var + 1e-5) * g_ref[...] + be_ref[...], 0.0)
        vn_ref[...] = vn
        vs_scr[pl.ds(j * BN, BN), :] = vn
        bm = jnp.max((vn + 1e-7) * st_ref[0, 1], axis=0, keepdims=True)

        @pl.when(j == 0)
        def _():
            gmax_scr[...] = bm

        @pl.when(j > 0)
        def _():
            gmax_scr[...] = jnp.maximum(gmax_scr[...], bm)

    @pl.when(j >= NBLK)
    def _():
        i = j - NBLK
        v = vs_scr[pl.ds(i * BN, BN), :]
        p0, p1 = _pay_from(v, gmax_scr[...], st_ref[0, 1])
        pay_ref[0] = p0
        pay_ref[1] = p1


def _post_mid(acc, v, hprev, w1, b1, mg, mb, w2, b2, g_next, b_next, si, tn):
    st = jnp.stack([si, tn]).reshape(1, 2)
    return pl.pallas_call(
        _post_mid_body,
        grid=(2 * NBLK,),
        in_specs=[
            pl.BlockSpec((1, BN, D), lambda i: (0, jnp.minimum(i, NBLK - 1), 0)),
            pl.BlockSpec((1, BN, D), lambda i: (1, jnp.minimum(i, NBLK - 1), 0)),
            pl.BlockSpec((BN, D), lambda i: (jnp.minimum(i, NBLK - 1), 0)),
            pl.BlockSpec((BN, D), lambda i: (jnp.minimum(i, NBLK - 1), 0)),
            pl.BlockSpec((D, H), lambda i: (0, 0)),
            pl.BlockSpec((1, H), lambda i: (0, 0)),
            pl.BlockSpec((1, H), lambda i: (0, 0)),
            pl.BlockSpec((1, H), lambda i: (0, 0)),
            pl.BlockSpec((H, D), lambda i: (0, 0)),
            pl.BlockSpec((1, D), lambda i: (0, 0)),
            pl.BlockSpec((1, D), lambda i: (0, 0)),
            pl.BlockSpec((1, D), lambda i: (0, 0)),
            pl.BlockSpec(memory_space=pltpu.SMEM),
        ],
        out_specs=[
            pl.BlockSpec((BN, D), lambda i: (jnp.minimum(i, NBLK - 1), 0)),
            pl.BlockSpec((BN, D), lambda i: (jnp.minimum(i, NBLK - 1), 0)),
            pl.BlockSpec((2, BN, D),
                         lambda i: (0, jnp.maximum(i - NBLK, 0), 0)),
        ],
        out_shape=[
            jax.ShapeDtypeStruct((N, D), jnp.float32),
            jax.ShapeDtypeStruct((N, D), jnp.float32),
            jax.ShapeDtypeStruct((2, N, D), jnp.float32),
        ],
        scratch_shapes=[
            pltpu.VMEM((N, D), jnp.float32),
            pltpu.VMEM((1, D), jnp.float32),
        ],
    )(acc, acc, v, hprev, w1, b1.reshape(1, H), mg.reshape(1, H),
      mb.reshape(1, H), w2, b2.reshape(1, D), g_next.reshape(1, D),
      b_next.reshape(1, D), st)


def _post_last_body(acc_ref0, acc_ref1, v_ref, hp_ref, w1_ref, b1_ref,
                    mg_ref, mb_ref, w2_ref, b2_ref, g_ref, be_ref, wo_ref,
                    bo_ref, st_ref, out_ref):
    h = _conv_tail(acc_ref0[0], acc_ref1[0], v_ref[...], hp_ref[...], w1_ref,
                   b1_ref, mg_ref, mb_ref, w2_ref, b2_ref, st_ref)
    mu = jnp.mean(h, axis=-1, keepdims=True)
    var = jnp.mean(jnp.square(h - mu), axis=-1, keepdims=True)
    z = jnp.maximum((h - mu) / jnp.sqrt(var + 1e-5) * g_ref[...] + be_ref[...], 0.0)
    out_ref[...] = jnp.dot(z, wo_ref[...], preferred_element_type=jnp.float32) + bo_ref[...]


def _post_last(acc, v, hprev, w1, b1, mg, mb, w2, b2, g0, b0, w_pad, bo_pad, si):
    st = jnp.stack([si, si]).reshape(1, 2)
    return pl.pallas_call(
        _post_last_body,
        grid=(N // BN,),
        in_specs=[
            pl.BlockSpec((1, BN, D), lambda i: (0, i, 0)),
            pl.BlockSpec((1, BN, D), lambda i: (1, i, 0)),
            pl.BlockSpec((BN, D), lambda i: (i, 0)),
            pl.BlockSpec((BN, D), lambda i: (i, 0)),
            pl.BlockSpec((D, H), lambda i: (0, 0)),
            pl.BlockSpec((1, H), lambda i: (0, 0)),
            pl.BlockSpec((1, H), lambda i: (0, 0)),
            pl.BlockSpec((1, H), lambda i: (0, 0)),
            pl.BlockSpec((H, D), lambda i: (0, 0)),
            pl.BlockSpec((1, D), lambda i: (0, 0)),
            pl.BlockSpec((1, D), lambda i: (0, 0)),
            pl.BlockSpec((1, D), lambda i: (0, 0)),
            pl.BlockSpec((D, D), lambda i: (0, 0)),
            pl.BlockSpec((1, D), lambda i: (0, 0)),
            pl.BlockSpec(memory_space=pltpu.SMEM),
        ],
        out_specs=pl.BlockSpec((BN, D), lambda i: (i, 0)),
        out_shape=jax.ShapeDtypeStruct((N, D), jnp.float32),
    )(acc, acc, v, hprev, w1, b1.reshape(1, H), mg.reshape(1, H),
      mb.reshape(1, H), w2, b2.reshape(1, D), g0.reshape(1, D),
      b0.reshape(1, D), w_pad, bo_pad, st)


def _edge_body(srcb_hbm, dst_hbm, pay_hbm, out_hbm,
               srcw_v, dstw_v, rows_v, acc_sh, semi, sem0, sem1):
    c = lax.axis_index("c")
    s = lax.axis_index("s")

    pltpu.async_copy(srcb_hbm.at[c, s, pl.ds(0, WCH)], srcw_v.at[0], semi)
    pltpu.async_copy(dst_hbm.at[s, pl.ds(0, WCH)], dstw_v.at[0], semi)

    zv = jnp.zeros((16,), jnp.float32)

    def zrow(r, _):
        for k in range(8):
            rows_v[0, r, pl.ds(k * 16, 16)] = zv
        return 0

    lax.fori_loop(0, CW, zrow, 0)
    for q in range(RPT // CW):
        pltpu.sync_copy(rows_v.at[0],
                        acc_sh.at[pl.ds(s * RPT + q * CW, CW)])
    rem = RPT - (RPT // CW) * CW
    pltpu.sync_copy(rows_v.at[0, pl.ds(0, rem)],
                    acc_sh.at[pl.ds(s * RPT + (RPT // CW) * CW, rem)])

    sems = (sem0, sem1)
    pltpu.make_async_copy(srcb_hbm.at[c, s, pl.ds(0, WCH)],
                          srcw_v.at[0], semi).wait()
    pltpu.make_async_copy(dst_hbm.at[s, pl.ds(0, WCH)],
                          dstw_v.at[0], semi).wait()
    for b in range(2):
        pltpu.async_copy(pay_hbm.at[srcw_v.at[0, b]], rows_v.at[b], sems[b])
    plsc.subcore_barrier()

    def wpair(k, _):
        for wb in range(2):
            w = 2 * k + wb

            @pl.when(w + 1 < NWIN)
            def _():
                pltpu.async_copy(srcb_hbm.at[c, s, pl.ds((w + 1) * WCH, WCH)],
                                 srcw_v.at[1 - wb], semi)
                pltpu.async_copy(dst_hbm.at[s, pl.ds((w + 1) * WCH, WCH)],
                                 dstw_v.at[1 - wb], semi)

            def cpair(kk, _):
                for b in range(2):
                    j = 2 * kk + b
                    pltpu.make_async_copy(pay_hbm.at[srcw_v.at[wb, j]],
                                          rows_v.at[b], sems[b]).wait()
                    pltpu.sync_copy(rows_v.at[b],
                                    acc_sh.at[dstw_v.at[wb, j]], add=True)

                    @pl.when(j + 2 < WCH)
                    def _():
                        pltpu.async_copy(pay_hbm.at[srcw_v.at[wb, j + 2]],
                                         rows_v.at[b], sems[b])
                return 0

            lax.fori_loop(0, WCH // 2, cpair, 0)

            @pl.when(w + 1 < NWIN)
            def _():
                pltpu.make_async_copy(
                    srcb_hbm.at[c, s, pl.ds((w + 1) * WCH, WCH)],
                    srcw_v.at[1 - wb], semi).wait()
                pltpu.make_async_copy(
                    dst_hbm.at[s, pl.ds((w + 1) * WCH, WCH)],
                    dstw_v.at[1 - wb], semi).wait()
                for b in range(2):
                    pltpu.async_copy(pay_hbm.at[srcw_v.at[1 - wb, b]],
                                     rows_v.at[b], sems[b])
        return 0

    lax.fori_loop(0, NWIN // 2, wpair, 0)
    plsc.subcore_barrier()
    pltpu.sync_copy(acc_sh.at[pl.ds(s * RPT, RPT)],
                    out_hbm.at[c, pl.ds(s * RPT, RPT)])


_edge_kernel = functools.partial(
    pl.kernel,
    _edge_body,
    out_type=jax.ShapeDtypeStruct((NC, NPAD, D), jnp.float32),
    mesh=plsc.VectorSubcoreMesh(core_axis_name="c", subcore_axis_name="s"),
    scratch_types=[
        pltpu.VMEM((2, WCH, CW), jnp.int32),
        pltpu.VMEM((2, WCH, CW), jnp.int32),
        pltpu.VMEM((2, CW, D), jnp.float32),
        pltpu.VMEM_SHARED((NPAD, D), jnp.float32),
        pltpu.SemaphoreType.DMA,
        pltpu.SemaphoreType.DMA,
        pltpu.SemaphoreType.DMA,
    ],
)()


def kernel(x, edge_index, W_enc, b_enc, ln_g, ln_b, t, W1, b1, mg, mb,
           W2, b2, scale, W_out, b_out):
    L = W1.shape[0]
    src = edge_index[0]
    dst = edge_index[1]
    npad = EPAD - E
    ar = jnp.arange(npad, dtype=jnp.int32)
    src_p = jnp.concatenate([src.astype(jnp.int32), ar % 128])
    dst_p = jnp.concatenate([dst.astype(jnp.int32), N + (ar % 16)])
    src_both = jnp.stack([src_p, src_p + N]).reshape(NC, NS, CHUNKS, CW)
    dst3 = dst_p.reshape(NS, CHUNKS, CW)
    hzero = jnp.zeros((N, D), jnp.float32)

    w_pad = jnp.zeros((D, D), jnp.float32).at[:, :C].set(W_out)
    bo_pad = jnp.zeros((1, D), jnp.float32).at[0, :C].set(b_out)

    h, pay = _enc(x, W_enc, b_enc, t[0])
    v = h
    for i in range(L):
        acc = _edge_kernel(src_both, dst3, pay.reshape(2 * N, D))
        hprev = hzero if i == 0 else h
        if i + 1 < L:
            h, v, pay = _post_mid(acc, v, hprev, W1[i], b1[i], mg[i], mb[i],
                                  W2[i], b2[i], ln_g[i + 1], ln_b[i + 1],
                                  scale[i], t[i + 1])
        else:
            out = _post_last(acc, v, hprev, W1[i], b1[i], mg[i], mb[i],
                             W2[i], b2[i], ln_g[0], ln_b[0], w_pad, bo_pad,
                             scale[i])
    return out[:, :C]


# speedup vs baseline: 1.0186x; 1.0186x over previous
import functools

import jax
import jax.numpy as jnp
from jax import lax
from jax.experimental import pallas as pl
from jax.experimental.pallas import tpu as pltpu
from jax.experimental.pallas import tpu_sc as plsc

N = 10000
E = 320000
D = 128
H = 256
C = 47

NC = 2
NS = 16
CW = 125
WCH = 16
NWIN = 10
CHUNKS = WCH * NWIN
EPAD = NS * CHUNKS * CW
NPAD = 10112
RPT = NPAD // NS

BN = 5000


NBLK = N // BN


def _pay_from(v, gmax, tv):
    m = jnp.maximum(v, 0.0) + 1e-7
    ex = jnp.exp(m * tv - gmax)
    mex = m * ex
    hd = D // 2
    return (jnp.concatenate([mex[:, :hd], ex[:, :hd]], axis=1),
            jnp.concatenate([mex[:, hd:], ex[:, hd:]], axis=1))


def _enc_body(x_ref, w_ref, b_ref, t_ref, h_ref, pay_ref, hs_scr, gmax_scr):
    j = pl.program_id(0)

    @pl.when(j < NBLK)
    def _():
        h = jnp.dot(x_ref[...], w_ref[...], preferred_element_type=jnp.float32)
        h = h + b_ref[...]
        h_ref[...] = h
        hs_scr[pl.ds(j * BN, BN), :] = h
        m = jnp.maximum(h, 0.0) + 1e-7
        bm = jnp.max(m * t_ref[0, 0], axis=0, keepdims=True)

        @pl.when(j == 0)
        def _():
            gmax_scr[...] = bm

        @pl.when(j > 0)
        def _():
            gmax_scr[...] = jnp.maximum(gmax_scr[...], bm)

    @pl.when(j >= NBLK)
    def _():
        i = j - NBLK
        v = hs_scr[pl.ds(i * BN, BN), :]
        p0, p1 = _pay_from(v, gmax_scr[...], t_ref[0, 0])
        pay_ref[0] = p0
        pay_ref[1] = p1


def _enc(x, w, b, t0):
    return pl.pallas_call(
        _enc_body,
        grid=(2 * NBLK,),
        in_specs=[
            pl.BlockSpec((BN, D), lambda i: (jnp.minimum(i, NBLK - 1), 0)),
            pl.BlockSpec((D, D), lambda i: (0, 0)),
            pl.BlockSpec((1, D), lambda i: (0, 0)),
            pl.BlockSpec(memory_space=pltpu.SMEM),
        ],
        out_specs=[
            pl.BlockSpec((BN, D), lambda i: (jnp.minimum(i, NBLK - 1), 0)),
            pl.BlockSpec((2, BN, D),
                         lambda i: (0, jnp.maximum(i - NBLK, 0), 0)),
        ],
        out_shape=[
            jax.ShapeDtypeStruct((N, D), jnp.float32),
            jax.ShapeDtypeStruct((2, N, D), jnp.float32),
        ],
        scratch_shapes=[
            pltpu.VMEM((N, D), jnp.float32),
            pltpu.VMEM((1, D), jnp.float32),
        ],
    )(x, w, b.reshape(1, D), t0.reshape(1, 1))


def _conv_tail(acc0, acc1, v, hp, w1_ref, b1_ref, mg_ref, mb_ref, w2_ref,
               b2_ref, s_ref):
    hd = D // 2
    num = jnp.concatenate([acc0[:, :hd], acc1[:, :hd]], axis=1)
    den = jnp.concatenate([acc0[:, hd:], acc1[:, hd:]], axis=1)
    agg = num / (den + 1e-16)
    msg = agg / jnp.sqrt(jnp.sum(agg * agg, axis=-1, keepdims=True) + 1e-24)
    xn = jnp.sqrt(jnp.sum(v * v, axis=-1, keepdims=True) + 1e-24)
    out = v + msg * xn * s_ref[0, 0]
    z = jnp.dot(out, w1_ref[...], preferred_element_type=jnp.float32)
    z = z + b1_ref[...]
    mu = jnp.mean(z, axis=-1, keepdims=True)
    var = jnp.mean(jnp.square(z - mu), axis=-1, keepdims=True)
    z = (z - mu) / jnp.sqrt(var + 1e-5) * mg_ref[...] + mb_ref[...]
    z = jnp.maximum(z, 0.0)
    r = jnp.dot(z, w2_ref[...], preferred_element_type=jnp.float32)
    return hp + r + b2_ref[...]


def _post_mid_body(acc_ref0, acc_ref1, v_ref, hp_ref, w1_ref, b1_ref, mg_ref,
                   mb_ref, w2_ref, b2_ref, g_ref, be_ref, st_ref,
                   h_ref, vn_ref, pay_ref, vs_scr, gmax_scr):
    j = pl.program_id(0)

    @pl.when(j < NBLK)
    def _():
        h = _conv_tail(acc_ref0[0], acc_ref1[0], v_ref[...], hp_ref[...],
                       w1_ref, b1_ref, mg_ref, mb_ref, w2_ref, b2_ref, st_ref)
        h_ref[...] = h
        mu = jnp.mean(h, axis=-1, keepdims=True)
        var = jnp.mean(jnp.square(h - mu), axis=-1, keepdims=True)
        vn = jnp.maximum(
            (h - mu) / jnp.sqrt(var + 1e-5) * g_ref[...] + be_ref[...], 0.0)
        vn_ref[...] = vn
        vs_scr[pl.ds(j * BN, BN), :] = vn
        bm = jnp.max((vn + 1e-7) * st_ref[0, 1], axis=0, keepdims=True)

        @pl.when(j == 0)
        def _():
            gmax_scr[...] = bm

        @pl.when(j > 0)
        def _():
            gmax_scr[...] = jnp.maximum(gmax_scr[...], bm)

    @pl.when(j >= NBLK)
    def _():
        i = j - NBLK
        v = vs_scr[pl.ds(i * BN, BN), :]
        p0, p1 = _pay_from(v, gmax_scr[...], st_ref[0, 1])
        pay_ref[0] = p0
        pay_ref[1] = p1


def _post_mid(acc, v, hprev, w1, b1, mg, mb, w2, b2, g_next, b_next, si, tn):
    st = jnp.stack([si, tn]).reshape(1, 2)
    return pl.pallas_call(
        _post_mid_body,
        grid=(2 * NBLK,),
        in_specs=[
            pl.BlockSpec((1, BN, D), lambda i: (0, jnp.minimum(i, NBLK - 1), 0)),
            pl.BlockSpec((1, BN, D), lambda i: (1, jnp.minimum(i, NBLK - 1), 0)),
            pl.BlockSpec((BN, D), lambda i: (jnp.minimum(i, NBLK - 1), 0)),
            pl.BlockSpec((BN, D), lambda i: (jnp.minimum(i, NBLK - 1), 0)),
            pl.BlockSpec((D, H), lambda i: (0, 0)),
            pl.BlockSpec((1, H), lambda i: (0, 0)),
            pl.BlockSpec((1, H), lambda i: (0, 0)),
            pl.BlockSpec((1, H), lambda i: (0, 0)),
            pl.BlockSpec((H, D), lambda i: (0, 0)),
            pl.BlockSpec((1, D), lambda i: (0, 0)),
            pl.BlockSpec((1, D), lambda i: (0, 0)),
            pl.BlockSpec((1, D), lambda i: (0, 0)),
            pl.BlockSpec(memory_space=pltpu.SMEM),
        ],
        out_specs=[
            pl.BlockSpec((BN, D), lambda i: (jnp.minimum(i, NBLK - 1), 0)),
            pl.BlockSpec((BN, D), lambda i: (jnp.minimum(i, NBLK - 1), 0)),
            pl.BlockSpec((2, BN, D),
                         lambda i: (0, jnp.maximum(i - NBLK, 0), 0)),
        ],
        out_shape=[
            jax.ShapeDtypeStruct((N, D), jnp.float32),
            jax.ShapeDtypeStruct((N, D), jnp.float32),
            jax.ShapeDtypeStruct((2, N, D), jnp.float32),
        ],
        scratch_shapes=[
            pltpu.VMEM((N, D), jnp.float32),
            pltpu.VMEM((1, D), jnp.float32),
        ],
    )(acc, acc, v, hprev, w1, b1.reshape(1, H), mg.reshape(1, H),
      mb.reshape(1, H), w2, b2.reshape(1, D), g_next.reshape(1, D),
      b_next.reshape(1, D), st)


def _post_last_body(acc_ref0, acc_ref1, v_ref, hp_ref, w1_ref, b1_ref,
                    mg_ref, mb_ref, w2_ref, b2_ref, g_ref, be_ref, wo_ref,
                    bo_ref, st_ref, out_ref):
    h = _conv_tail(acc_ref0[0], acc_ref1[0], v_ref[...], hp_ref[...], w1_ref,
                   b1_ref, mg_ref, mb_ref, w2_ref, b2_ref, st_ref)
    mu = jnp.mean(h, axis=-1, keepdims=True)
    var = jnp.mean(jnp.square(h - mu), axis=-1, keepdims=True)
    z = jnp.maximum((h - mu) / jnp.sqrt(var + 1e-5) * g_ref[...] + be_ref[...], 0.0)
    out_ref[...] = jnp.dot(z, wo_ref[...], preferred_element_type=jnp.float32) + bo_ref[...]


def _post_last(acc, v, hprev, w1, b1, mg, mb, w2, b2, g0, b0, w_pad, bo_pad, si):
    st = jnp.stack([si, si]).reshape(1, 2)
    return pl.pallas_call(
        _post_last_body,
        grid=(N // BN,),
        in_specs=[
            pl.BlockSpec((1, BN, D), lambda i: (0, i, 0)),
            pl.BlockSpec((1, BN, D), lambda i: (1, i, 0)),
            pl.BlockSpec((BN, D), lambda i: (i, 0)),
            pl.BlockSpec((BN, D), lambda i: (i, 0)),
            pl.BlockSpec((D, H), lambda i: (0, 0)),
            pl.BlockSpec((1, H), lambda i: (0, 0)),
            pl.BlockSpec((1, H), lambda i: (0, 0)),
            pl.BlockSpec((1, H), lambda i: (0, 0)),
            pl.BlockSpec((H, D), lambda i: (0, 0)),
            pl.BlockSpec((1, D), lambda i: (0, 0)),
            pl.BlockSpec((1, D), lambda i: (0, 0)),
            pl.BlockSpec((1, D), lambda i: (0, 0)),
            pl.BlockSpec((D, D), lambda i: (0, 0)),
            pl.BlockSpec((1, D), lambda i: (0, 0)),
            pl.BlockSpec(memory_space=pltpu.SMEM),
        ],
        out_specs=pl.BlockSpec((BN, D), lambda i: (i, 0)),
        out_shape=jax.ShapeDtypeStruct((N, D), jnp.float32),
    )(acc, acc, v, hprev, w1, b1.reshape(1, H), mg.reshape(1, H),
      mb.reshape(1, H), w2, b2.reshape(1, D), g0.reshape(1, D),
      b0.reshape(1, D), w_pad, bo_pad, st)


def _edge_body(srcb_hbm, dst_hbm, pay_hbm, out_hbm,
               srcw_v, dstw_v, rows_v, acc_sh, semi, sem0, sem1):
    c = lax.axis_index("c")
    s = lax.axis_index("s")

    pltpu.async_copy(srcb_hbm.at[c, s, pl.ds(0, WCH)], srcw_v.at[0], semi)
    pltpu.async_copy(dst_hbm.at[s, pl.ds(0, WCH)], dstw_v.at[0], semi)

    zv = jnp.zeros((16,), jnp.float32)

    def zrow(r, _):
        for k in range(8):
            rows_v[0, r, pl.ds(k * 16, 16)] = zv
        return 0

    lax.fori_loop(0, CW, zrow, 0)
    for q in range(RPT // CW):
        pltpu.sync_copy(rows_v.at[0],
                        acc_sh.at[pl.ds(s * RPT + q * CW, CW)])
    rem = RPT - (RPT // CW) * CW
    pltpu.sync_copy(rows_v.at[0, pl.ds(0, rem)],
                    acc_sh.at[pl.ds(s * RPT + (RPT // CW) * CW, rem)])

    sems = (sem0, sem1)
    pltpu.make_async_copy(srcb_hbm.at[c, s, pl.ds(0, WCH)],
                          srcw_v.at[0], semi).wait()
    pltpu.make_async_copy(dst_hbm.at[s, pl.ds(0, WCH)],
                          dstw_v.at[0], semi).wait()
    for b in range(2):
        pltpu.async_copy(pay_hbm.at[srcw_v.at[0, b]], rows_v.at[b], sems[b])
    plsc.subcore_barrier()

    def wpair(k, _):
        for wb in range(2):
            w = 2 * k + wb

            @pl.when(w + 1 < NWIN)
            def _():
                pltpu.async_copy(srcb_hbm.at[c, s, pl.ds((w + 1) * WCH, WCH)],
                                 srcw_v.at[1 - wb], semi)
                pltpu.async_copy(dst_hbm.at[s, pl.ds((w + 1) * WCH, WCH)],
                                 dstw_v.at[1 - wb], semi)

            def cpair(kk, _):
                for b in range(2):
                    j = 2 * kk + b
                    pltpu.make_async_copy(pay_hbm.at[srcw_v.at[wb, j]],
                                          rows_v.at[b], sems[b]).wait()
                    pltpu.sync_copy(rows_v.at[b],
                                    acc_sh.at[dstw_v.at[wb, j]], add=True)

                    @pl.when(j + 2 < WCH)
                    def _():
                        pltpu.async_copy(pay_hbm.at[srcw_v.at[wb, j + 2]],
                                         rows_v.at[b], sems[b])
                return 0

            lax.fori_loop(0, WCH // 2, cpair, 0)

            @pl.when(w + 1 < NWIN)
            def _():
                pltpu.make_async_copy(
                    srcb_hbm.at[c, s, pl.ds((w + 1) * WCH, WCH)],
                    srcw_v.at[1 - wb], semi).wait()
                pltpu.make_async_copy(
                    dst_hbm.at[s, pl.ds((w + 1) * WCH, WCH)],
                    dstw_v.at[1 - wb], semi).wait()
                for b in range(2):
                    pltpu.async_copy(pay_hbm.at[srcw_v.at[1 - wb, b]],
                                     rows_v.at[b], sems[b])
        return 0

    lax.fori_loop(0, NWIN // 2, wpair, 0)
    plsc.subcore_barrier()
    pltpu.sync_copy(acc_sh.at[pl.ds(s * RPT, RPT)],
                    out_hbm.at[c, pl.ds(s * RPT, RPT)])


_edge_kernel = functools.partial(
    pl.kernel,
    _edge_body,
    out_type=jax.ShapeDtypeStruct((NC, NPAD, D), jnp.float32),
    mesh=plsc.VectorSubcoreMesh(core_axis_name="c", subcore_axis_name="s"),
    scratch_types=[
        pltpu.VMEM((2, WCH, CW), jnp.int32),
        pltpu.VMEM((2, WCH, CW), jnp.int32),
        pltpu.VMEM((2, CW, D), jnp.float32),
        pltpu.VMEM_SHARED((NPAD, D), jnp.float32),
        pltpu.SemaphoreType.DMA,
        pltpu.SemaphoreType.DMA,
        pltpu.SemaphoreType.DMA,
    ],
)()


def kernel(x, edge_index, W_enc, b_enc, ln_g, ln_b, t, W1, b1, mg, mb,
           W2, b2, scale, W_out, b_out):
    L = W1.shape[0]
    src = edge_index[0]
    dst = edge_index[1]
    npad = EPAD - E
    ar = jnp.arange(npad, dtype=jnp.int32)
    src_p = jnp.concatenate([src.astype(jnp.int32), ar % 128])
    dst_p = jnp.concatenate([dst.astype(jnp.int32), N + (ar % 16)])
    src_both = jnp.stack([src_p, src_p + N]).reshape(NC, NS, CHUNKS, CW)
    dst3 = dst_p.reshape(NS, CHUNKS, CW)
    hzero = jnp.zeros((N, D), jnp.float32)

    w_pad = jnp.zeros((D, D), jnp.float32).at[:, :C].set(W_out)
    bo_pad = jnp.zeros((1, D), jnp.float32).at[0, :C].set(b_out)

    h, pay = _enc(x, W_enc, b_enc, t[0])
    v = h
    for i in range(L):
        acc = _edge_kernel(src_both, dst3, pay.reshape(2 * N, D))
        hprev = hzero if i == 0 else h
        if i + 1 < L:
            h, v, pay = _post_mid(acc, v, hprev, W1[i], b1[i], mg[i], mb[i],
                                  W2[i], b2[i], ln_g[i + 1], ln_b[i + 1],
                                  scale[i], t[i + 1])
        else:
            out = _post_last(acc, v, hprev, W1[i], b1[i], mg[i], mb[i],
                             W2[i], b2[i], ln_g[0], ln_b[0], w_pad, bo_pad,
                             scale[i])
    return out[:, :C]


# speedup vs baseline: 1.0284x; 1.0097x over previous
import functools

import jax
import jax.numpy as jnp
from jax import lax
from jax.experimental import pallas as pl
from jax.experimental.pallas import tpu as pltpu
from jax.experimental.pallas import tpu_sc as plsc

N = 10000
E = 320000
D = 128
H = 256
C = 47

NC = 2
NS = 16
CW = 125
WCH = 16
NWIN = 10
CHUNKS = WCH * NWIN
EPAD = NS * CHUNKS * CW
NPAD = 10112
RPT = NPAD // NS

BN = 2000


NBLK = N // BN


def _pay_from(v, gmax, tv):
    m = jnp.maximum(v, 0.0) + 1e-7
    ex = jnp.exp(m * tv - gmax)
    mex = m * ex
    hd = D // 2
    return (jnp.concatenate([mex[:, :hd], ex[:, :hd]], axis=1),
            jnp.concatenate([mex[:, hd:], ex[:, hd:]], axis=1))


def _enc_body(x_ref, w_ref, b_ref, t_ref, h_ref, pay_ref, hs_scr, gmax_scr):
    j = pl.program_id(0)

    @pl.when(j < NBLK)
    def _():
        h = jnp.dot(x_ref[...], w_ref[...], preferred_element_type=jnp.float32)
        h = h + b_ref[...]
        h_ref[...] = h
        hs_scr[pl.ds(j * BN, BN), :] = h
        m = jnp.maximum(h, 0.0) + 1e-7
        bm = jnp.max(m * t_ref[0, 0], axis=0, keepdims=True)

        @pl.when(j == 0)
        def _():
            gmax_scr[...] = bm

        @pl.when(j > 0)
        def _():
            gmax_scr[...] = jnp.maximum(gmax_scr[...], bm)

    @pl.when(j >= NBLK)
    def _():
        i = j - NBLK
        v = hs_scr[pl.ds(i * BN, BN), :]
        p0, p1 = _pay_from(v, gmax_scr[...], t_ref[0, 0])
        pay_ref[0] = p0
        pay_ref[1] = p1


def _enc(x, w, b, t0):
    return pl.pallas_call(
        _enc_body,
        grid=(2 * NBLK,),
        in_specs=[
            pl.BlockSpec((BN, D), lambda i: (jnp.minimum(i, NBLK - 1), 0)),
            pl.BlockSpec((D, D), lambda i: (0, 0)),
            pl.BlockSpec((1, D), lambda i: (0, 0)),
            pl.BlockSpec(memory_space=pltpu.SMEM),
        ],
        out_specs=[
            pl.BlockSpec((BN, D), lambda i: (jnp.minimum(i, NBLK - 1), 0)),
            pl.BlockSpec((2, BN, D),
                         lambda i: (0, jnp.maximum(i - NBLK, 0), 0)),
        ],
        out_shape=[
            jax.ShapeDtypeStruct((N, D), jnp.float32),
            jax.ShapeDtypeStruct((2, N, D), jnp.float32),
        ],
        scratch_shapes=[
            pltpu.VMEM((N, D), jnp.float32),
            pltpu.VMEM((1, D), jnp.float32),
        ],
    )(x, w, b.reshape(1, D), t0.reshape(1, 1))


def _conv_tail(acc0, acc1, v, hp, w1_ref, b1_ref, mg_ref, mb_ref, w2_ref,
               b2_ref, s_ref):
    hd = D // 2
    num = jnp.concatenate([acc0[:, :hd], acc1[:, :hd]], axis=1)
    den = jnp.concatenate([acc0[:, hd:], acc1[:, hd:]], axis=1)
    agg = num / (den + 1e-16)
    msg = agg / jnp.sqrt(jnp.sum(agg * agg, axis=-1, keepdims=True) + 1e-24)
    xn = jnp.sqrt(jnp.sum(v * v, axis=-1, keepdims=True) + 1e-24)
    out = v + msg * xn * s_ref[0, 0]
    z = jnp.dot(out, w1_ref[...], preferred_element_type=jnp.float32)
    z = z + b1_ref[...]
    mu = jnp.mean(z, axis=-1, keepdims=True)
    var = jnp.mean(jnp.square(z - mu), axis=-1, keepdims=True)
    z = (z - mu) / jnp.sqrt(var + 1e-5) * mg_ref[...] + mb_ref[...]
    z = jnp.maximum(z, 0.0)
    r = jnp.dot(z, w2_ref[...], preferred_element_type=jnp.float32)
    return hp + r + b2_ref[...]


def _post_mid_body(acc_ref0, acc_ref1, v_ref, hp_ref, w1_ref, b1_ref, mg_ref,
                   mb_ref, w2_ref, b2_ref, g_ref, be_ref, st_ref,
                   h_ref, vn_ref, pay_ref, vs_scr, gmax_scr):
    j = pl.program_id(0)

    @pl.when(j < NBLK)
    def _():
        h = _conv_tail(acc_ref0[0], acc_ref1[0], v_ref[...], hp_ref[...],
                       w1_ref, b1_ref, mg_ref, mb_ref, w2_ref, b2_ref, st_ref)
        h_ref[...] = h
        mu = jnp.mean(h, axis=-1, keepdims=True)
        var = jnp.mean(jnp.square(h - mu), axis=-1, keepdims=True)
        vn = jnp.maximum(
            (h - mu) / jnp.sqrt(var + 1e-5) * g_ref[...] + be_ref[...], 0.0)
        vn_ref[...] = vn
        vs_scr[pl.ds(j * BN, BN), :] = vn
        bm = jnp.max((vn + 1e-7) * st_ref[0, 1], axis=0, keepdims=True)

        @pl.when(j == 0)
        def _():
            gmax_scr[...] = bm

        @pl.when(j > 0)
        def _():
            gmax_scr[...] = jnp.maximum(gmax_scr[...], bm)

    @pl.when(j >= NBLK)
    def _():
        i = j - NBLK
        v = vs_scr[pl.ds(i * BN, BN), :]
        p0, p1 = _pay_from(v, gmax_scr[...], st_ref[0, 1])
        pay_ref[0] = p0
        pay_ref[1] = p1


def _post_mid(acc, v, hprev, w1, b1, mg, mb, w2, b2, g_next, b_next, si, tn):
    st = jnp.stack([si, tn]).reshape(1, 2)
    return pl.pallas_call(
        _post_mid_body,
        grid=(2 * NBLK,),
        in_specs=[
            pl.BlockSpec((1, BN, D), lambda i: (0, jnp.minimum(i, NBLK - 1), 0)),
            pl.BlockSpec((1, BN, D), lambda i: (1, jnp.minimum(i, NBLK - 1), 0)),
            pl.BlockSpec((BN, D), lambda i: (jnp.minimum(i, NBLK - 1), 0)),
            pl.BlockSpec((BN, D), lambda i: (jnp.minimum(i, NBLK - 1), 0)),
            pl.BlockSpec((D, H), lambda i: (0, 0)),
            pl.BlockSpec((1, H), lambda i: (0, 0)),
            pl.BlockSpec((1, H), lambda i: (0, 0)),
            pl.BlockSpec((1, H), lambda i: (0, 0)),
            pl.BlockSpec((H, D), lambda i: (0, 0)),
            pl.BlockSpec((1, D), lambda i: (0, 0)),
            pl.BlockSpec((1, D), lambda i: (0, 0)),
            pl.BlockSpec((1, D), lambda i: (0, 0)),
            pl.BlockSpec(memory_space=pltpu.SMEM),
        ],
        out_specs=[
            pl.BlockSpec((BN, D), lambda i: (jnp.minimum(i, NBLK - 1), 0)),
            pl.BlockSpec((BN, D), lambda i: (jnp.minimum(i, NBLK - 1), 0)),
            pl.BlockSpec((2, BN, D),
                         lambda i: (0, jnp.maximum(i - NBLK, 0), 0)),
        ],
        out_shape=[
            jax.ShapeDtypeStruct((N, D), jnp.float32),
            jax.ShapeDtypeStruct((N, D), jnp.float32),
            jax.ShapeDtypeStruct((2, N, D), jnp.float32),
        ],
        scratch_shapes=[
            pltpu.VMEM((N, D), jnp.float32),
            pltpu.VMEM((1, D), jnp.float32),
        ],
    )(acc, acc, v, hprev, w1, b1.reshape(1, H), mg.reshape(1, H),
      mb.reshape(1, H), w2, b2.reshape(1, D), g_next.reshape(1, D),
      b_next.reshape(1, D), st)


def _post_last_body(acc_ref0, acc_ref1, v_ref, hp_ref, w1_ref, b1_ref,
                    mg_ref, mb_ref, w2_ref, b2_ref, g_ref, be_ref, wo_ref,
                    bo_ref, st_ref, out_ref):
    h = _conv_tail(acc_ref0[0], acc_ref1[0], v_ref[...], hp_ref[...], w1_ref,
                   b1_ref, mg_ref, mb_ref, w2_ref, b2_ref, st_ref)
    mu = jnp.mean(h, axis=-1, keepdims=True)
    var = jnp.mean(jnp.square(h - mu), axis=-1, keepdims=True)
    z = jnp.maximum((h - mu) / jnp.sqrt(var + 1e-5) * g_ref[...] + be_ref[...], 0.0)
    out_ref[...] = jnp.dot(z, wo_ref[...], preferred_element_type=jnp.float32) + bo_ref[...]


def _post_last(acc, v, hprev, w1, b1, mg, mb, w2, b2, g0, b0, w_pad, bo_pad, si):
    st = jnp.stack([si, si]).reshape(1, 2)
    return pl.pallas_call(
        _post_last_body,
        grid=(N // BN,),
        in_specs=[
            pl.BlockSpec((1, BN, D), lambda i: (0, i, 0)),
            pl.BlockSpec((1, BN, D), lambda i: (1, i, 0)),
            pl.BlockSpec((BN, D), lambda i: (i, 0)),
            pl.BlockSpec((BN, D), lambda i: (i, 0)),
            pl.BlockSpec((D, H), lambda i: (0, 0)),
            pl.BlockSpec((1, H), lambda i: (0, 0)),
            pl.BlockSpec((1, H), lambda i: (0, 0)),
            pl.BlockSpec((1, H), lambda i: (0, 0)),
            pl.BlockSpec((H, D), lambda i: (0, 0)),
            pl.BlockSpec((1, D), lambda i: (0, 0)),
            pl.BlockSpec((1, D), lambda i: (0, 0)),
            pl.BlockSpec((1, D), lambda i: (0, 0)),
            pl.BlockSpec((D, D), lambda i: (0, 0)),
            pl.BlockSpec((1, D), lambda i: (0, 0)),
            pl.BlockSpec(memory_space=pltpu.SMEM),
        ],
        out_specs=pl.BlockSpec((BN, D), lambda i: (i, 0)),
        out_shape=jax.ShapeDtypeStruct((N, D), jnp.float32),
    )(acc, acc, v, hprev, w1, b1.reshape(1, H), mg.reshape(1, H),
      mb.reshape(1, H), w2, b2.reshape(1, D), g0.reshape(1, D),
      b0.reshape(1, D), w_pad, bo_pad, st)


def _edge_body(srcb_hbm, dst_hbm, pay_hbm, out_hbm,
               srcw_v, dstw_v, rows_v, acc_sh, semi, sem0, sem1):
    c = lax.axis_index("c")
    s = lax.axis_index("s")

    pltpu.async_copy(srcb_hbm.at[s, pl.ds(0, WCH)], srcw_v.at[0], semi)
    pltpu.async_copy(dst_hbm.at[s, pl.ds(0, WCH)], dstw_v.at[0], semi)

    zv = jnp.zeros((16,), jnp.float32)

    def zrow(r, _):
        for k in range(8):
            rows_v[0, r, pl.ds(k * 16, 16)] = zv
        return 0

    lax.fori_loop(0, CW, zrow, 0)
    for q in range(RPT // CW):
        pltpu.sync_copy(rows_v.at[0],
                        acc_sh.at[pl.ds(s * RPT + q * CW, CW)])
    rem = RPT - (RPT // CW) * CW
    pltpu.sync_copy(rows_v.at[0, pl.ds(0, rem)],
                    acc_sh.at[pl.ds(s * RPT + (RPT // CW) * CW, rem)])

    sems = (sem0, sem1)
    pltpu.make_async_copy(srcb_hbm.at[s, pl.ds(0, WCH)],
                          srcw_v.at[0], semi).wait()
    pltpu.make_async_copy(dst_hbm.at[s, pl.ds(0, WCH)],
                          dstw_v.at[0], semi).wait()
    for b in range(2):
        pltpu.async_copy(pay_hbm.at[c].at[srcw_v.at[0, b]], rows_v.at[b], sems[b])
    plsc.subcore_barrier()

    def wpair(k, _):
        for wb in range(2):
            w = 2 * k + wb

            @pl.when(w + 1 < NWIN)
            def _():
                pltpu.async_copy(srcb_hbm.at[s, pl.ds((w + 1) * WCH, WCH)],
                                 srcw_v.at[1 - wb], semi)
                pltpu.async_copy(dst_hbm.at[s, pl.ds((w + 1) * WCH, WCH)],
                                 dstw_v.at[1 - wb], semi)

            def cpair(kk, _):
                for b in range(2):
                    j = 2 * kk + b
                    pltpu.make_async_copy(pay_hbm.at[c].at[srcw_v.at[wb, j]],
                                          rows_v.at[b], sems[b]).wait()
                    pltpu.sync_copy(rows_v.at[b],
                                    acc_sh.at[dstw_v.at[wb, j]], add=True)

                    @pl.when(j + 2 < WCH)
                    def _():
                        pltpu.async_copy(pay_hbm.at[c].at[srcw_v.at[wb, j + 2]],
                                         rows_v.at[b], sems[b])
                return 0

            lax.fori_loop(0, WCH // 2, cpair, 0)

            @pl.when(w + 1 < NWIN)
            def _():
                pltpu.make_async_copy(
                    srcb_hbm.at[s, pl.ds((w + 1) * WCH, WCH)],
                    srcw_v.at[1 - wb], semi).wait()
                pltpu.make_async_copy(
                    dst_hbm.at[s, pl.ds((w + 1) * WCH, WCH)],
                    dstw_v.at[1 - wb], semi).wait()
                for b in range(2):
                    pltpu.async_copy(pay_hbm.at[c].at[srcw_v.at[1 - wb, b]],
                                     rows_v.at[b], sems[b])
        return 0

    lax.fori_loop(0, NWIN // 2, wpair, 0)
    plsc.subcore_barrier()
    pltpu.sync_copy(acc_sh.at[pl.ds(s * RPT, RPT)],
                    out_hbm.at[c, pl.ds(s * RPT, RPT)])


_edge_kernel = functools.partial(
    pl.kernel,
    _edge_body,
    out_type=jax.ShapeDtypeStruct((NC, NPAD, D), jnp.float32),
    mesh=plsc.VectorSubcoreMesh(core_axis_name="c", subcore_axis_name="s"),
    scratch_types=[
        pltpu.VMEM((2, WCH, CW), jnp.int32),
        pltpu.VMEM((2, WCH, CW), jnp.int32),
        pltpu.VMEM((2, CW, D), jnp.float32),
        pltpu.VMEM_SHARED((NPAD, D), jnp.float32),
        pltpu.SemaphoreType.DMA,
        pltpu.SemaphoreType.DMA,
        pltpu.SemaphoreType.DMA,
    ],
)()


def kernel(x, edge_index, W_enc, b_enc, ln_g, ln_b, t, W1, b1, mg, mb,
           W2, b2, scale, W_out, b_out):
    L = W1.shape[0]
    src3 = edge_index[0].astype(jnp.int32).reshape(NS, CHUNKS, CW)
    dst3 = edge_index[1].astype(jnp.int32).reshape(NS, CHUNKS, CW)
    hzero = jnp.zeros((N, D), jnp.float32)

    w_pad = jnp.zeros((D, D), jnp.float32).at[:, :C].set(W_out)
    bo_pad = jnp.zeros((1, D), jnp.float32).at[0, :C].set(b_out)

    h, pay = _enc(x, W_enc, b_enc, t[0])
    v = h
    for i in range(L):
        acc = _edge_kernel(src3, dst3, pay)
        hprev = hzero if i == 0 else h
        if i + 1 < L:
            h, v, pay = _post_mid(acc, v, hprev, W1[i], b1[i], mg[i], mb[i],
                                  W2[i], b2[i], ln_g[i + 1], ln_b[i + 1],
                                  scale[i], t[i + 1])
        else:
            out = _post_last(acc, v, hprev, W1[i], b1[i], mg[i], mb[i],
                             W2[i], b2[i], ln_g[0], ln_b[0], w_pad, bo_pad,
                             scale[i])
    return out[:, :C]


# speedup vs baseline: 1.0288x; 1.0004x over previous
import functools

import jax
import jax.numpy as jnp
from jax import lax
from jax.experimental import pallas as pl
from jax.experimental.pallas import tpu as pltpu
from jax.experimental.pallas import tpu_sc as plsc

N = 10000
E = 320000
D = 128
H = 256
C = 47

NC = 2
NS = 16
CW = 125
WCH = 16
NWIN = 10
CHUNKS = WCH * NWIN
EPAD = NS * CHUNKS * CW
NPAD = 10112
RPT = NPAD // NS

BN = 2000


NBLK = N // BN


def _pay_from(v, gmax, tv):
    m = jnp.maximum(v, 0.0) + 1e-7
    ex = jnp.exp(m * tv - gmax)
    mex = m * ex
    hd = D // 2
    return (jnp.concatenate([mex[:, :hd], ex[:, :hd]], axis=1),
            jnp.concatenate([mex[:, hd:], ex[:, hd:]], axis=1))


def _enc_body(x_ref, w_ref, b_ref, t_ref, h_ref, pay_ref, hs_scr, gmax_scr):
    j = pl.program_id(0)

    @pl.when(j < NBLK)
    def _():
        h = jnp.dot(x_ref[...], w_ref[...], preferred_element_type=jnp.float32)
        h = h + b_ref[...]
        h_ref[...] = h
        hs_scr[pl.ds(j * BN, BN), :] = h
        m = jnp.maximum(h, 0.0) + 1e-7
        bm = jnp.max(m * t_ref[0, 0], axis=0, keepdims=True)

        @pl.when(j == 0)
        def _():
            gmax_scr[...] = bm

        @pl.when(j > 0)
        def _():
            gmax_scr[...] = jnp.maximum(gmax_scr[...], bm)

    @pl.when(j >= NBLK)
    def _():
        i = j - NBLK
        v = hs_scr[pl.ds(i * BN, BN), :]
        p0, p1 = _pay_from(v, gmax_scr[...], t_ref[0, 0])
        pay_ref[0] = p0
        pay_ref[1] = p1


def _enc(x, w, b, t0):
    return pl.pallas_call(
        _enc_body,
        grid=(2 * NBLK,),
        in_specs=[
            pl.BlockSpec((BN, D), lambda i: (jnp.minimum(i, NBLK - 1), 0)),
            pl.BlockSpec((D, D), lambda i: (0, 0)),
            pl.BlockSpec((1, D), lambda i: (0, 0)),
            pl.BlockSpec(memory_space=pltpu.SMEM),
        ],
        out_specs=[
            pl.BlockSpec((BN, D), lambda i: (jnp.minimum(i, NBLK - 1), 0)),
            pl.BlockSpec((2, BN, D),
                         lambda i: (0, jnp.maximum(i - NBLK, 0), 0)),
        ],
        out_shape=[
            jax.ShapeDtypeStruct((N, D), jnp.float32),
            jax.ShapeDtypeStruct((2, N, D), jnp.float32),
        ],
        scratch_shapes=[
            pltpu.VMEM((N, D), jnp.float32),
            pltpu.VMEM((1, D), jnp.float32),
        ],
    )(x, w, b.reshape(1, D), t0.reshape(1, 1))


def _conv_tail(acc0, acc1, v, hp, w1_ref, b1_ref, mg_ref, mb_ref, w2_ref,
               b2_ref, s_ref):
    hd = D // 2
    num = jnp.concatenate([acc0[:, :hd], acc1[:, :hd]], axis=1)
    den = jnp.concatenate([acc0[:, hd:], acc1[:, hd:]], axis=1)
    agg = num / (den + 1e-16)
    msg = agg / jnp.sqrt(jnp.sum(agg * agg, axis=-1, keepdims=True) + 1e-24)
    xn = jnp.sqrt(jnp.sum(v * v, axis=-1, keepdims=True) + 1e-24)
    out = v + msg * xn * s_ref[0, 0]
    z = jnp.dot(out, w1_ref[...], preferred_element_type=jnp.float32)
    z = z + b1_ref[...]
    mu = jnp.mean(z, axis=-1, keepdims=True)
    var = jnp.mean(jnp.square(z - mu), axis=-1, keepdims=True)
    z = (z - mu) / jnp.sqrt(var + 1e-5) * mg_ref[...] + mb_ref[...]
    z = jnp.maximum(z, 0.0)
    r = jnp.dot(z, w2_ref[...], preferred_element_type=jnp.float32)
    return hp + r + b2_ref[...]


def _post_mid_body(acc_ref0, acc_ref1, v_ref, hp_ref, w1_ref, b1_ref, mg_ref,
                   mb_ref, w2_ref, b2_ref, g_ref, be_ref, st_ref,
                   h_ref, vn_ref, pay_ref, vs_scr, gmax_scr):
    j = pl.program_id(0)

    @pl.when(j < NBLK)
    def _():
        h = _conv_tail(acc_ref0[0], acc_ref1[0], v_ref[...], hp_ref[...],
                       w1_ref, b1_ref, mg_ref, mb_ref, w2_ref, b2_ref, st_ref)
        h_ref[...] = h
        mu = jnp.mean(h, axis=-1, keepdims=True)
        var = jnp.mean(jnp.square(h - mu), axis=-1, keepdims=True)
        vn = jnp.maximum(
            (h - mu) / jnp.sqrt(var + 1e-5) * g_ref[...] + be_ref[...], 0.0)
        vn_ref[...] = vn
        vs_scr[pl.ds(j * BN, BN), :] = vn
        bm = jnp.max((vn + 1e-7) * st_ref[0, 1], axis=0, keepdims=True)

        @pl.when(j == 0)
        def _():
            gmax_scr[...] = bm

        @pl.when(j > 0)
        def _():
            gmax_scr[...] = jnp.maximum(gmax_scr[...], bm)

    @pl.when(j >= NBLK)
    def _():
        i = j - NBLK
        v = vs_scr[pl.ds(i * BN, BN), :]
        p0, p1 = _pay_from(v, gmax_scr[...], st_ref[0, 1])
        pay_ref[0] = p0
        pay_ref[1] = p1


def _post_mid(acc, v, hprev, w1, b1, mg, mb, w2, b2, g_next, b_next, si, tn):
    st = jnp.stack([si, tn]).reshape(1, 2)
    return pl.pallas_call(
        _post_mid_body,
        grid=(2 * NBLK,),
        in_specs=[
            pl.BlockSpec((1, BN, D), lambda i: (0, jnp.minimum(i, NBLK - 1), 0)),
            pl.BlockSpec((1, BN, D), lambda i: (1, jnp.minimum(i, NBLK - 1), 0)),
            pl.BlockSpec((BN, D), lambda i: (jnp.minimum(i, NBLK - 1), 0)),
            pl.BlockSpec((BN, D), lambda i: (jnp.minimum(i, NBLK - 1), 0)),
            pl.BlockSpec((D, H), lambda i: (0, 0)),
            pl.BlockSpec((1, H), lambda i: (0, 0)),
            pl.BlockSpec((1, H), lambda i: (0, 0)),
            pl.BlockSpec((1, H), lambda i: (0, 0)),
            pl.BlockSpec((H, D), lambda i: (0, 0)),
            pl.BlockSpec((1, D), lambda i: (0, 0)),
            pl.BlockSpec((1, D), lambda i: (0, 0)),
            pl.BlockSpec((1, D), lambda i: (0, 0)),
            pl.BlockSpec(memory_space=pltpu.SMEM),
        ],
        out_specs=[
            pl.BlockSpec((BN, D), lambda i: (jnp.minimum(i, NBLK - 1), 0)),
            pl.BlockSpec((BN, D), lambda i: (jnp.minimum(i, NBLK - 1), 0)),
            pl.BlockSpec((2, BN, D),
                         lambda i: (0, jnp.maximum(i - NBLK, 0), 0)),
        ],
        out_shape=[
            jax.ShapeDtypeStruct((N, D), jnp.float32),
            jax.ShapeDtypeStruct((N, D), jnp.float32),
            jax.ShapeDtypeStruct((2, N, D), jnp.float32),
        ],
        scratch_shapes=[
            pltpu.VMEM((N, D), jnp.float32),
            pltpu.VMEM((1, D), jnp.float32),
        ],
    )(acc, acc, v, hprev, w1, b1.reshape(1, H), mg.reshape(1, H),
      mb.reshape(1, H), w2, b2.reshape(1, D), g_next.reshape(1, D),
      b_next.reshape(1, D), st)


def _post_last_body(acc_ref0, acc_ref1, v_ref, hp_ref, w1_ref, b1_ref,
                    mg_ref, mb_ref, w2_ref, b2_ref, g_ref, be_ref, wo_ref,
                    bo_ref, st_ref, out_ref):
    h = _conv_tail(acc_ref0[0], acc_ref1[0], v_ref[...], hp_ref[...], w1_ref,
                   b1_ref, mg_ref, mb_ref, w2_ref, b2_ref, st_ref)
    mu = jnp.mean(h, axis=-1, keepdims=True)
    var = jnp.mean(jnp.square(h - mu), axis=-1, keepdims=True)
    z = jnp.maximum((h - mu) / jnp.sqrt(var + 1e-5) * g_ref[...] + be_ref[...], 0.0)
    y = jnp.dot(z, wo_ref[...], preferred_element_type=jnp.float32) + bo_ref[...]
    out_ref[...] = y[:, :C]


def _post_last(acc, v, hprev, w1, b1, mg, mb, w2, b2, g0, b0, w_pad, bo_pad, si):
    st = jnp.stack([si, si]).reshape(1, 2)
    return pl.pallas_call(
        _post_last_body,
        grid=(N // BN,),
        in_specs=[
            pl.BlockSpec((1, BN, D), lambda i: (0, i, 0)),
            pl.BlockSpec((1, BN, D), lambda i: (1, i, 0)),
            pl.BlockSpec((BN, D), lambda i: (i, 0)),
            pl.BlockSpec((BN, D), lambda i: (i, 0)),
            pl.BlockSpec((D, H), lambda i: (0, 0)),
            pl.BlockSpec((1, H), lambda i: (0, 0)),
            pl.BlockSpec((1, H), lambda i: (0, 0)),
            pl.BlockSpec((1, H), lambda i: (0, 0)),
            pl.BlockSpec((H, D), lambda i: (0, 0)),
            pl.BlockSpec((1, D), lambda i: (0, 0)),
            pl.BlockSpec((1, D), lambda i: (0, 0)),
            pl.BlockSpec((1, D), lambda i: (0, 0)),
            pl.BlockSpec((D, D), lambda i: (0, 0)),
            pl.BlockSpec((1, D), lambda i: (0, 0)),
            pl.BlockSpec(memory_space=pltpu.SMEM),
        ],
        out_specs=pl.BlockSpec((BN, C), lambda i: (i, 0)),
        out_shape=jax.ShapeDtypeStruct((N, C), jnp.float32),
    )(acc, acc, v, hprev, w1, b1.reshape(1, H), mg.reshape(1, H),
      mb.reshape(1, H), w2, b2.reshape(1, D), g0.reshape(1, D),
      b0.reshape(1, D), w_pad, bo_pad, st)


def _edge_body(srcb_hbm, dst_hbm, pay_hbm, out_hbm,
               srcw_v, dstw_v, rows_v, acc_sh, semi, sem0, sem1):
    c = lax.axis_index("c")
    s = lax.axis_index("s")

    pltpu.async_copy(srcb_hbm.at[s, pl.ds(0, WCH)], srcw_v.at[0], semi)
    pltpu.async_copy(dst_hbm.at[s, pl.ds(0, WCH)], dstw_v.at[0], semi)

    zv = jnp.zeros((16,), jnp.float32)

    def zrow(r, _):
        for k in range(8):
            rows_v[0, r, pl.ds(k * 16, 16)] = zv
        return 0

    lax.fori_loop(0, CW, zrow, 0)
    for q in range(RPT // CW):
        pltpu.sync_copy(rows_v.at[0],
                        acc_sh.at[pl.ds(s * RPT + q * CW, CW)])
    rem = RPT - (RPT // CW) * CW
    pltpu.sync_copy(rows_v.at[0, pl.ds(0, rem)],
                    acc_sh.at[pl.ds(s * RPT + (RPT // CW) * CW, rem)])

    sems = (sem0, sem1)
    pltpu.make_async_copy(srcb_hbm.at[s, pl.ds(0, WCH)],
                          srcw_v.at[0], semi).wait()
    pltpu.make_async_copy(dst_hbm.at[s, pl.ds(0, WCH)],
                          dstw_v.at[0], semi).wait()
    for b in range(2):
        pltpu.async_copy(pay_hbm.at[c].at[srcw_v.at[0, b]], rows_v.at[b], sems[b])
    plsc.subcore_barrier()

    def wpair(k, _):
        for wb in range(2):
            w = 2 * k + wb

            @pl.when(w + 1 < NWIN)
            def _():
                pltpu.async_copy(srcb_hbm.at[s, pl.ds((w + 1) * WCH, WCH)],
                                 srcw_v.at[1 - wb], semi)
                pltpu.async_copy(dst_hbm.at[s, pl.ds((w + 1) * WCH, WCH)],
                                 dstw_v.at[1 - wb], semi)

            def cpair(kk, _):
                for b in range(2):
                    j = 2 * kk + b
                    pltpu.make_async_copy(pay_hbm.at[c].at[srcw_v.at[wb, j]],
                                          rows_v.at[b], sems[b]).wait()
                    pltpu.sync_copy(rows_v.at[b],
                                    acc_sh.at[dstw_v.at[wb, j]], add=True)

                    @pl.when(j + 2 < WCH)
                    def _():
                        pltpu.async_copy(pay_hbm.at[c].at[srcw_v.at[wb, j + 2]],
                                         rows_v.at[b], sems[b])
                return 0

            lax.fori_loop(0, WCH // 2, cpair, 0)

            @pl.when(w + 1 < NWIN)
            def _():
                pltpu.make_async_copy(
                    srcb_hbm.at[s, pl.ds((w + 1) * WCH, WCH)],
                    srcw_v.at[1 - wb], semi).wait()
                pltpu.make_async_copy(
                    dst_hbm.at[s, pl.ds((w + 1) * WCH, WCH)],
                    dstw_v.at[1 - wb], semi).wait()
                for b in range(2):
                    pltpu.async_copy(pay_hbm.at[c].at[srcw_v.at[1 - wb, b]],
                                     rows_v.at[b], sems[b])
        return 0

    lax.fori_loop(0, NWIN // 2, wpair, 0)
    plsc.subcore_barrier()
    pltpu.sync_copy(acc_sh.at[pl.ds(s * RPT, RPT)],
                    out_hbm.at[c, pl.ds(s * RPT, RPT)])


_edge_kernel = functools.partial(
    pl.kernel,
    _edge_body,
    out_type=jax.ShapeDtypeStruct((NC, NPAD, D), jnp.float32),
    mesh=plsc.VectorSubcoreMesh(core_axis_name="c", subcore_axis_name="s"),
    scratch_types=[
        pltpu.VMEM((2, WCH, CW), jnp.int32),
        pltpu.VMEM((2, WCH, CW), jnp.int32),
        pltpu.VMEM((2, CW, D), jnp.float32),
        pltpu.VMEM_SHARED((NPAD, D), jnp.float32),
        pltpu.SemaphoreType.DMA,
        pltpu.SemaphoreType.DMA,
        pltpu.SemaphoreType.DMA,
    ],
)()


def kernel(x, edge_index, W_enc, b_enc, ln_g, ln_b, t, W1, b1, mg, mb,
           W2, b2, scale, W_out, b_out):
    L = W1.shape[0]
    src3 = edge_index[0].astype(jnp.int32).reshape(NS, CHUNKS, CW)
    dst3 = edge_index[1].astype(jnp.int32).reshape(NS, CHUNKS, CW)
    hzero = jnp.zeros((N, D), jnp.float32)

    w_pad = jnp.zeros((D, D), jnp.float32).at[:, :C].set(W_out)
    bo_pad = jnp.zeros((1, D), jnp.float32).at[0, :C].set(b_out)

    h, pay = _enc(x, W_enc, b_enc, t[0])
    v = h
    for i in range(L):
        acc = _edge_kernel(src3, dst3, pay)
        hprev = hzero if i == 0 else h
        if i + 1 < L:
            h, v, pay = _post_mid(acc, v, hprev, W1[i], b1[i], mg[i], mb[i],
                                  W2[i], b2[i], ln_g[i + 1], ln_b[i + 1],
                                  scale[i], t[i + 1])
        else:
            out = _post_last(acc, v, hprev, W1[i], b1[i], mg[i], mb[i],
                             W2[i], b2[i], ln_g[0], ln_b[0], w_pad, bo_pad,
                             scale[i])
    return out


# speedup vs baseline: 1.0399x; 1.0108x over previous
import functools

import jax
import jax.numpy as jnp
from jax import lax
from jax.experimental import pallas as pl
from jax.experimental.pallas import tpu as pltpu
from jax.experimental.pallas import tpu_sc as plsc

N = 10000
E = 320000
D = 128
H = 256
C = 47

NC = 2
NS = 16
CW = 125
WCH = 16
NWIN = 10
CHUNKS = WCH * NWIN
EPAD = NS * CHUNKS * CW
NPAD = 10112
RPT = NPAD // NS

BN = 2000


NBLK = N // BN


def _pay_from(v, gmax, tv):
    m = jnp.maximum(v, 0.0) + 1e-7
    ex = jnp.exp(m * tv - gmax)
    mex = m * ex
    hd = D // 2
    return (jnp.concatenate([mex[:, :hd], ex[:, :hd]], axis=1),
            jnp.concatenate([mex[:, hd:], ex[:, hd:]], axis=1))


def _enc_body(x_ref, w_ref, b_ref, t_ref, h_ref, pay_ref, hs_scr, gmax_scr):
    j = pl.program_id(0)

    @pl.when(j < NBLK)
    def _():
        h = jnp.dot(x_ref[...], w_ref[...], preferred_element_type=jnp.float32)
        h = h + b_ref[...]
        h_ref[...] = h
        hs_scr[pl.ds(j * BN, BN), :] = h
        m = jnp.maximum(h, 0.0) + 1e-7
        bm = jnp.max(m * t_ref[0, 0], axis=0, keepdims=True)

        @pl.when(j == 0)
        def _():
            gmax_scr[...] = bm

        @pl.when(j > 0)
        def _():
            gmax_scr[...] = jnp.maximum(gmax_scr[...], bm)

    @pl.when(j >= NBLK)
    def _():
        i = j - NBLK
        v = hs_scr[pl.ds(i * BN, BN), :]
        p0, p1 = _pay_from(v, gmax_scr[...], t_ref[0, 0])
        pay_ref[0] = p0
        pay_ref[1] = p1


def _enc(x, w, b, t0):
    return pl.pallas_call(
        _enc_body,
        grid=(2 * NBLK,),
        in_specs=[
            pl.BlockSpec((BN, D), lambda i: (jnp.minimum(i, NBLK - 1), 0)),
            pl.BlockSpec((D, D), lambda i: (0, 0)),
            pl.BlockSpec((1, D), lambda i: (0, 0)),
            pl.BlockSpec(memory_space=pltpu.SMEM),
        ],
        out_specs=[
            pl.BlockSpec((BN, D), lambda i: (jnp.minimum(i, NBLK - 1), 0)),
            pl.BlockSpec((2, BN, D),
                         lambda i: (0, jnp.maximum(i - NBLK, 0), 0)),
        ],
        out_shape=[
            jax.ShapeDtypeStruct((N, D), jnp.float32),
            jax.ShapeDtypeStruct((2, N, D), jnp.float32),
        ],
        scratch_shapes=[
            pltpu.VMEM((N, D), jnp.float32),
            pltpu.VMEM((1, D), jnp.float32),
        ],
    )(x, w, b.reshape(1, D), t0.reshape(1, 1))


def _conv_tail(acc0, acc1, v, hp, w1_ref, b1_ref, mg_ref, mb_ref, w2_ref,
               b2_ref, s_ref):
    hd = D // 2
    num = jnp.concatenate([acc0[:, :hd], acc1[:, :hd]], axis=1)
    den = jnp.concatenate([acc0[:, hd:], acc1[:, hd:]], axis=1)
    agg = num / (den + 1e-16)
    msg = agg / jnp.sqrt(jnp.sum(agg * agg, axis=-1, keepdims=True) + 1e-24)
    xn = jnp.sqrt(jnp.sum(v * v, axis=-1, keepdims=True) + 1e-24)
    out = v + msg * xn * s_ref[0, 0]
    z = jnp.dot(out, w1_ref[...], preferred_element_type=jnp.float32)
    z = z + b1_ref[...]
    mu = jnp.mean(z, axis=-1, keepdims=True)
    var = jnp.mean(jnp.square(z - mu), axis=-1, keepdims=True)
    z = (z - mu) / jnp.sqrt(var + 1e-5) * mg_ref[...] + mb_ref[...]
    z = jnp.maximum(z, 0.0)
    r = jnp.dot(z, w2_ref[...], preferred_element_type=jnp.float32)
    return hp + r + b2_ref[...]


def _post_mid_body(acc_ref0, acc_ref1, v_ref, hp_ref, w1_ref, b1_ref, mg_ref,
                   mb_ref, w2_ref, b2_ref, g_ref, be_ref, st_ref,
                   h_ref, vn_ref, pay_ref, vs_scr, gmax_scr):
    j = pl.program_id(0)

    @pl.when(j < NBLK)
    def _():
        h = _conv_tail(acc_ref0[0], acc_ref1[0], v_ref[...], hp_ref[...],
                       w1_ref, b1_ref, mg_ref, mb_ref, w2_ref, b2_ref, st_ref)
        h_ref[...] = h
        mu = jnp.mean(h, axis=-1, keepdims=True)
        var = jnp.mean(jnp.square(h - mu), axis=-1, keepdims=True)
        vn = jnp.maximum(
            (h - mu) / jnp.sqrt(var + 1e-5) * g_ref[...] + be_ref[...], 0.0)
        vn_ref[...] = vn
        vs_scr[pl.ds(j * BN, BN), :] = vn
        bm = jnp.max((vn + 1e-7) * st_ref[0, 1], axis=0, keepdims=True)

        @pl.when(j == 0)
        def _():
            gmax_scr[...] = bm

        @pl.when(j > 0)
        def _():
            gmax_scr[...] = jnp.maximum(gmax_scr[...], bm)

    @pl.when(j >= NBLK)
    def _():
        i = j - NBLK
        v = vs_scr[pl.ds(i * BN, BN), :]
        p0, p1 = _pay_from(v, gmax_scr[...], st_ref[0, 1])
        pay_ref[0] = p0
        pay_ref[1] = p1


def _post_mid(acc, v, hprev, w1, b1, mg, mb, w2, b2, g_next, b_next, si, tn):
    st = jnp.stack([si, tn]).reshape(1, 2)
    return pl.pallas_call(
        _post_mid_body,
        grid=(2 * NBLK,),
        in_specs=[
            pl.BlockSpec((1, BN, D), lambda i: (0, jnp.minimum(i, NBLK - 1), 0)),
            pl.BlockSpec((1, BN, D), lambda i: (1, jnp.minimum(i, NBLK - 1), 0)),
            pl.BlockSpec((BN, D), lambda i: (jnp.minimum(i, NBLK - 1), 0)),
            pl.BlockSpec((BN, D), lambda i: (jnp.minimum(i, NBLK - 1), 0)),
            pl.BlockSpec((D, H), lambda i: (0, 0)),
            pl.BlockSpec((1, H), lambda i: (0, 0)),
            pl.BlockSpec((1, H), lambda i: (0, 0)),
            pl.BlockSpec((1, H), lambda i: (0, 0)),
            pl.BlockSpec((H, D), lambda i: (0, 0)),
            pl.BlockSpec((1, D), lambda i: (0, 0)),
            pl.BlockSpec((1, D), lambda i: (0, 0)),
            pl.BlockSpec((1, D), lambda i: (0, 0)),
            pl.BlockSpec(memory_space=pltpu.SMEM),
        ],
        out_specs=[
            pl.BlockSpec((BN, D), lambda i: (jnp.minimum(i, NBLK - 1), 0)),
            pl.BlockSpec((BN, D), lambda i: (jnp.minimum(i, NBLK - 1), 0)),
            pl.BlockSpec((2, BN, D),
                         lambda i: (0, jnp.maximum(i - NBLK, 0), 0)),
        ],
        out_shape=[
            jax.ShapeDtypeStruct((N, D), jnp.float32),
            jax.ShapeDtypeStruct((N, D), jnp.float32),
            jax.ShapeDtypeStruct((2, N, D), jnp.float32),
        ],
        scratch_shapes=[
            pltpu.VMEM((N, D), jnp.float32),
            pltpu.VMEM((1, D), jnp.float32),
        ],
    )(acc, acc, v, hprev, w1, b1.reshape(1, H), mg.reshape(1, H),
      mb.reshape(1, H), w2, b2.reshape(1, D), g_next.reshape(1, D),
      b_next.reshape(1, D), st)


def _post_last_body(acc_ref0, acc_ref1, v_ref, hp_ref, w1_ref, b1_ref,
                    mg_ref, mb_ref, w2_ref, b2_ref, g_ref, be_ref, wo_ref,
                    bo_ref, st_ref, out_ref):
    h = _conv_tail(acc_ref0[0], acc_ref1[0], v_ref[...], hp_ref[...], w1_ref,
                   b1_ref, mg_ref, mb_ref, w2_ref, b2_ref, st_ref)
    mu = jnp.mean(h, axis=-1, keepdims=True)
    var = jnp.mean(jnp.square(h - mu), axis=-1, keepdims=True)
    z = jnp.maximum((h - mu) / jnp.sqrt(var + 1e-5) * g_ref[...] + be_ref[...], 0.0)
    y = jnp.dot(z, wo_ref[...], preferred_element_type=jnp.float32) + bo_ref[...]
    out_ref[...] = y[:, :C]


def _post_last(acc, v, hprev, w1, b1, mg, mb, w2, b2, g0, b0, w_pad, bo_pad, si):
    st = jnp.stack([si, si]).reshape(1, 2)
    return pl.pallas_call(
        _post_last_body,
        grid=(N // BN,),
        in_specs=[
            pl.BlockSpec((1, BN, D), lambda i: (0, i, 0)),
            pl.BlockSpec((1, BN, D), lambda i: (1, i, 0)),
            pl.BlockSpec((BN, D), lambda i: (i, 0)),
            pl.BlockSpec((BN, D), lambda i: (i, 0)),
            pl.BlockSpec((D, H), lambda i: (0, 0)),
            pl.BlockSpec((1, H), lambda i: (0, 0)),
            pl.BlockSpec((1, H), lambda i: (0, 0)),
            pl.BlockSpec((1, H), lambda i: (0, 0)),
            pl.BlockSpec((H, D), lambda i: (0, 0)),
            pl.BlockSpec((1, D), lambda i: (0, 0)),
            pl.BlockSpec((1, D), lambda i: (0, 0)),
            pl.BlockSpec((1, D), lambda i: (0, 0)),
            pl.BlockSpec((D, D), lambda i: (0, 0)),
            pl.BlockSpec((1, D), lambda i: (0, 0)),
            pl.BlockSpec(memory_space=pltpu.SMEM),
        ],
        out_specs=pl.BlockSpec((BN, C), lambda i: (i, 0)),
        out_shape=jax.ShapeDtypeStruct((N, C), jnp.float32),
    )(acc, acc, v, hprev, w1, b1.reshape(1, H), mg.reshape(1, H),
      mb.reshape(1, H), w2, b2.reshape(1, D), g0.reshape(1, D),
      b0.reshape(1, D), w_pad, bo_pad, st)


def _edge_body(srcb_hbm, pay_hbm, out_hbm,
               srcw_v, dstw_v, rows_v, acc_sh, semi, sem0, sem1):
    c = lax.axis_index("c")
    s = lax.axis_index("s")

    pltpu.async_copy(srcb_hbm.at[0, s, pl.ds(0, WCH), :], srcw_v.at[0], semi)
    pltpu.async_copy(srcb_hbm.at[1, s, pl.ds(0, WCH), :], dstw_v.at[0], semi)

    zv = jnp.zeros((16,), jnp.float32)

    def zrow(r, _):
        for k in range(8):
            rows_v[0, r, pl.ds(k * 16, 16)] = zv
        return 0

    lax.fori_loop(0, CW, zrow, 0)
    for q in range(RPT // CW):
        pltpu.sync_copy(rows_v.at[0],
                        acc_sh.at[pl.ds(s * RPT + q * CW, CW)])
    rem = RPT - (RPT // CW) * CW
    pltpu.sync_copy(rows_v.at[0, pl.ds(0, rem)],
                    acc_sh.at[pl.ds(s * RPT + (RPT // CW) * CW, rem)])

    sems = (sem0, sem1)
    pltpu.make_async_copy(srcb_hbm.at[0, s, pl.ds(0, WCH), :],
                          srcw_v.at[0], semi).wait()
    pltpu.make_async_copy(srcb_hbm.at[1, s, pl.ds(0, WCH), :],
                          dstw_v.at[0], semi).wait()
    for b in range(2):
        pltpu.async_copy(pay_hbm.at[c].at[srcw_v.at[0, b]], rows_v.at[b], sems[b])
    plsc.subcore_barrier()

    def wpair(k, _):
        for wb in range(2):
            w = 2 * k + wb

            @pl.when(w + 1 < NWIN)
            def _():
                pltpu.async_copy(
                    srcb_hbm.at[0, s, pl.ds((w + 1) * WCH, WCH), :],
                    srcw_v.at[1 - wb], semi)
                pltpu.async_copy(
                    srcb_hbm.at[1, s, pl.ds((w + 1) * WCH, WCH), :],
                    dstw_v.at[1 - wb], semi)

            def cpair(kk, _):
                for b in range(2):
                    j = 2 * kk + b
                    pltpu.make_async_copy(pay_hbm.at[c].at[srcw_v.at[wb, j]],
                                          rows_v.at[b], sems[b]).wait()
                    pltpu.sync_copy(rows_v.at[b],
                                    acc_sh.at[dstw_v.at[wb, j]], add=True)

                    @pl.when(j + 2 < WCH)
                    def _():
                        pltpu.async_copy(pay_hbm.at[c].at[srcw_v.at[wb, j + 2]],
                                         rows_v.at[b], sems[b])
                return 0

            lax.fori_loop(0, WCH // 2, cpair, 0)

            @pl.when(w + 1 < NWIN)
            def _():
                pltpu.make_async_copy(
                    srcb_hbm.at[0, s, pl.ds((w + 1) * WCH, WCH), :],
                    srcw_v.at[1 - wb], semi).wait()
                pltpu.make_async_copy(
                    srcb_hbm.at[1, s, pl.ds((w + 1) * WCH, WCH), :],
                    dstw_v.at[1 - wb], semi).wait()
                for b in range(2):
                    pltpu.async_copy(pay_hbm.at[c].at[srcw_v.at[1 - wb, b]],
                                     rows_v.at[b], sems[b])
        return 0

    lax.fori_loop(0, NWIN // 2, wpair, 0)
    plsc.subcore_barrier()
    pltpu.sync_copy(acc_sh.at[pl.ds(s * RPT, RPT)],
                    out_hbm.at[c, pl.ds(s * RPT, RPT)])


_edge_kernel = functools.partial(
    pl.kernel,
    _edge_body,
    out_type=jax.ShapeDtypeStruct((NC, NPAD, D), jnp.float32),
    mesh=plsc.VectorSubcoreMesh(core_axis_name="c", subcore_axis_name="s"),
    scratch_types=[
        pltpu.VMEM((2, WCH, CW), jnp.int32),
        pltpu.VMEM((2, WCH, CW), jnp.int32),
        pltpu.VMEM((2, CW, D), jnp.float32),
        pltpu.VMEM_SHARED((NPAD, D), jnp.float32),
        pltpu.SemaphoreType.DMA,
        pltpu.SemaphoreType.DMA,
        pltpu.SemaphoreType.DMA,
    ],
)()


def kernel(x, edge_index, W_enc, b_enc, ln_g, ln_b, t, W1, b1, mg, mb,
           W2, b2, scale, W_out, b_out):
    L = W1.shape[0]
    srcdst = edge_index.astype(jnp.int32).reshape(2, NS, CHUNKS, CW)
    hzero = jnp.zeros((N, D), jnp.float32)

    w_pad = jnp.zeros((D, D), jnp.float32).at[:, :C].set(W_out)
    bo_pad = jnp.zeros((1, D), jnp.float32).at[0, :C].set(b_out)

    h, pay = _enc(x, W_enc, b_enc, t[0])
    v = h
    for i in range(L):
        acc = _edge_kernel(srcdst, pay)
        hprev = hzero if i == 0 else h
        if i + 1 < L:
            h, v, pay = _post_mid(acc, v, hprev, W1[i], b1[i], mg[i], mb[i],
                                  W2[i], b2[i], ln_g[i + 1], ln_b[i + 1],
                                  scale[i], t[i + 1])
        else:
            out = _post_last(acc, v, hprev, W1[i], b1[i], mg[i], mb[i],
                             W2[i], b2[i], ln_g[0], ln_b[0], w_pad, bo_pad,
                             scale[i])
    return out
